# baseline (speedup 1.0000x reference)
import functools

import jax
import jax.numpy as jnp
from jax import lax
from jax.experimental import pallas as pl
from jax.experimental.pallas import tpu as pltpu
from jax.experimental.pallas import tpu_sc as plsc

NC = 2
NS = 16
NW = NC * NS

SB = 1024
NIDX = 128

QW = 512
NQ_PW = 63
NQ = NW * NQ_PW
NROW_PAD = NQ * 4 * 128


@functools.lru_cache(maxsize=None)
def _make_detile(num_emb: int, emb_dim: int):
    assert emb_dim == 32
    n_full = num_emb // 128
    tail_rows = num_emb - n_full * 128
    tail_q = n_full // 4
    clamp = (num_emb - QW) // 128 * 128
    assert n_full % 4 == 0 and tail_rows == 64 and NQ * 4 > n_full

    mesh = plsc.VectorSubcoreMesh(core_axis_name="c", subcore_axis_name="s")

    @functools.partial(
        pl.kernel,
        mesh=mesh,
        compiler_params=pltpu.CompilerParams(
            use_tc_tiling_on_sc=True, needs_layout_passes=False),
        out_type=jax.ShapeDtypeStruct((NQ * 4 * 32, 128), jnp.float32),
        scratch_types=[
            pltpu.VMEM((emb_dim, QW), jnp.float32),
            pltpu.VMEM((emb_dim, QW), jnp.float32),
            pltpu.VMEM((128, 128), jnp.float32),
            pltpu.VMEM((128, 128), jnp.float32),
            pltpu.VMEM((emb_dim, 64), jnp.float32),
            pltpu.SemaphoreType.DMA,
            pltpu.SemaphoreType.DMA,
            pltpu.SemaphoreType.DMA,
            pltpu.SemaphoreType.DMA,
        ],
    )
    def detile(tblT, o_hbm, tin_a, tin_b, tout_a, tout_b, tin64,
               is_a, is_b, os_a, os_b):
        wid = lax.axis_index("s") * NC + lax.axis_index("c")
        iota16 = lax.iota(jnp.int32, 16)

        def q_of(c):
            return c * NW + wid

        def fire(c, tin, sem):
            s = jnp.minimum(q_of(c) * QW, clamp)
            pltpu.async_copy(
                tblT.at[pl.ds(0, emb_dim), pl.ds(s, QW)], tin, sem)

        def drain_in(tin, sem):
            pltpu.make_async_copy(
                tblT.at[pl.ds(0, emb_dim), pl.ds(0, QW)], tin, sem).wait()

        def drain_out(tout, sem):
            pltpu.make_async_copy(
                tout, o_hbm.at[pl.ds(0, 128)], sem).wait()

        def transpose_quad(tin, tout):
            @plsc.parallel_loop(0, 32, 1, unroll=4)
            def step(h2):
                sl = h2 // 8
                rl0 = lax.rem(h2, 8)
                rlvec = rl0 * 16 + iota16
                rl_in = sl * 128 + rlvec
                rowvec = sl * 32 + lax.shift_right_logical(rlvec, 2)
                colbase = jnp.bitwise_and(rlvec, 3) * emb_dim
                for d0 in range(emb_dim):
                    dvec = jnp.bitwise_and(d0 + iota16, emb_dim - 1)
                    plsc.store_scatter(
                        tout,
                        [rowvec, colbase + dvec],
                        plsc.load_gather(tin, [dvec, rl_in]),
                    )

        def proc(c, tin, tout, osem):
            q = q_of(c)
            transpose_quad(tin, tout)

            @pl.when(q == tail_q)
            def _():
                pltpu.sync_copy(
                    tblT.at[pl.ds(0, emb_dim), pl.ds(n_full * 128, 64)],
                    tin64)

                @plsc.parallel_loop(0, 4, 1)
                def tstep(rl0):
                    rlvec = rl0 * 16 + iota16
                    rowvec = lax.shift_right_logical(rlvec, 2)
                    colbase = jnp.bitwise_and(rlvec, 3) * emb_dim
                    for d0 in range(emb_dim):
                        dvec = jnp.bitwise_and(d0 + iota16, emb_dim - 1)
                        plsc.store_scatter(
                            tout,
                            [rowvec, colbase + dvec],
                            plsc.load_gather(tin64, [dvec, rlvec]),
                        )

            pltpu.async_copy(tout, o_hbm.at[pl.ds(q * 128, 128)], osem)

        fire(0, tin_a, is_a)

        def pair(h, carry):
            c = 2 * h
            fire(c + 1, tin_b, is_b)
            drain_in(tin_a, is_a)

            @pl.when(h > 0)
            def _():
                drain_out(tout_a, os_a)

            proc(c, tin_a, tout_a, os_a)
            fire(c + 2, tin_a, is_a)
            drain_in(tin_b, is_b)

            @pl.when(h > 0)
            def _():
                drain_out(tout_b, os_b)

            proc(c + 1, tin_b, tout_b, os_b)
            return carry

        lax.fori_loop(0, (NQ_PW - 1) // 2, pair, 0)
        drain_in(tin_a, is_a)
        drain_out(tout_a, os_a)
        proc(NQ_PW - 1, tin_a, tout_a, os_a)
        drain_out(tout_a, os_a)
        drain_out(tout_b, os_b)

    return detile


@functools.lru_cache(maxsize=None)
def _make_emb(n_l: int, n_b: int, emb_dim: int):
    ntok = n_l * n_b
    assert emb_dim == 32 and n_b % 128 == 0 and ntok % (NW * SB) == 0
    pw = ntok // NW
    nsb = pw // SB
    assert nsb % 2 == 1 and nsb >= 3
    npairs = (nsb - 1) // 2
    sb_per_l = n_b // SB

    mesh = plsc.VectorSubcoreMesh(core_axis_name="c", subcore_axis_name="s")

    @functools.partial(
        pl.kernel,
        mesh=mesh,
        compiler_params=pltpu.CompilerParams(
            use_tc_tiling_on_sc=False, needs_layout_passes=False),
        out_type=jax.ShapeDtypeStruct(
            (n_l, emb_dim // 8, n_b // 128, 8, 128), jnp.float32),
        scratch_types=[
            pltpu.VMEM((pw,), jnp.int32),
            pltpu.VMEM((SB, emb_dim), jnp.float32),
            pltpu.VMEM((SB, emb_dim), jnp.float32),
            pltpu.VMEM((emb_dim // 8, SB // 128, 8, 128), jnp.float32),
            pltpu.SemaphoreType.DMA,
            pltpu.SemaphoreType.DMA,
            pltpu.SemaphoreType.DMA,
        ],
    )
    def emb(tbl, idx_hbm, p_hbm, idx_v, rows_a, rows_b, tr,
            gs_a, gs_b, osem):
        wid = lax.axis_index("s") * NC + lax.axis_index("c")
        base = wid * pw
        pltpu.sync_copy(idx_hbm.at[pl.ds(base, pw)], idx_v)
        sb0 = wid * nsb

        def fire(i, rows, sem):
            for k in range(SB // NIDX):
                pltpu.async_copy(
                    tbl.at[idx_v.at[pl.ds(i * SB + k * NIDX, NIDX)]],
                    rows.at[pl.ds(k * NIDX, NIDX)],
                    sem,
                )

        def drain(rows, sem):
            pltpu.make_async_copy(tbl.at[pl.ds(0, SB)], rows, sem).wait()

        def drain_out():
            for dg in range(emb_dim // 8):
                pltpu.make_async_copy(
                    tr.at[dg], p_hbm.at[0, dg, pl.ds(0, SB // 128)], osem,
                ).wait()

        def process(i, rows):
            @pl.when(i > 0)
            def _():
                drain_out()
            iota16 = lax.iota(jnp.int32, 16)

            @plsc.parallel_loop(0, SB // 16, 1, unroll=4)
            def tconv(j):
                ridx = j * 16 + iota16
                bgvec = jnp.full((16,), j // 8, jnp.int32)
                blvec = lax.rem(j, 8) * 16 + iota16
                for d0 in range(emb_dim):
                    dvec = jnp.bitwise_and(d0 + iota16, emb_dim - 1)
                    v = plsc.load_gather(rows, [ridx, dvec])
                    plsc.store_scatter(
                        tr,
                        [lax.shift_right_logical(dvec, 3), bgvec,
                         jnp.bitwise_and(dvec, 7), blvec],
                        v,
                    )
            sb = sb0 + i
            l = sb // sb_per_l
            bgbase = lax.rem(sb, sb_per_l) * (SB // 128)
            for dg in range(emb_dim // 8):
                pltpu.async_copy(
                    tr.at[dg],
                    p_hbm.at[l, dg, pl.ds(bgbase, SB // 128)],
                    osem,
                )

        fire(0, rows_a, gs_a)

        def pair(h, carry):
            i = 2 * h
            fire(i + 1, rows_b, gs_b)
            drain(rows_a, gs_a)
            process(i, rows_a)
            fire(i + 2, rows_a, gs_a)
            drain(rows_b, gs_b)
            process(i + 1, rows_b)
            return carry

        lax.fori_loop(0, npairs, pair, 0)
        drain(rows_a, gs_a)
        process(nsb - 1, rows_a)
        drain_out()

    return emb


def kernel(x, table):
    b, l = x.shape
    num_emb, emb_dim = table.shape
    idx = x.T.reshape(-1).astype(jnp.int32)
    o = _make_detile(num_emb, emb_dim)(table.T)
    tbl_lin = o.reshape(NROW_PAD, emb_dim)
    p = _make_emb(l, b, emb_dim)(tbl_lin, idx)
    return p.transpose(2, 4, 0, 1, 3).reshape(b, l, emb_dim)


# speedup vs baseline: 1.2452x; 1.2452x over previous
import functools

import jax
import jax.numpy as jnp
from jax import lax
from jax.experimental import pallas as pl
from jax.experimental.pallas import tpu as pltpu
from jax.experimental.pallas import tpu_sc as plsc

NC = 2
NS = 16
NW = NC * NS

SB = 1024
NIDX = 128

QW = 512
NQ_PW = 63
NQ = NW * NQ_PW
NROW_PAD = NQ * 4 * 128


@functools.lru_cache(maxsize=None)
def _make_detile(num_emb: int, emb_dim: int):
    assert emb_dim == 32
    n_full = num_emb // 128
    tail_rows = num_emb - n_full * 128
    tail_q = n_full // 4
    clamp = (num_emb - QW) // 128 * 128
    assert n_full % 4 == 0 and tail_rows == 64 and NQ * 4 > n_full

    mesh = plsc.VectorSubcoreMesh(core_axis_name="c", subcore_axis_name="s")

    @functools.partial(
        pl.kernel,
        mesh=mesh,
        compiler_params=pltpu.CompilerParams(
            use_tc_tiling_on_sc=True, needs_layout_passes=False),
        out_type=jax.ShapeDtypeStruct((NQ * 4 * 32, 128), jnp.float32),
        scratch_types=[
            pltpu.VMEM((emb_dim, QW), jnp.float32),
            pltpu.VMEM((emb_dim, QW), jnp.float32),
            pltpu.VMEM((128, 128), jnp.float32),
            pltpu.VMEM((128, 128), jnp.float32),
            pltpu.VMEM((emb_dim, 64), jnp.float32),
            pltpu.SemaphoreType.DMA,
            pltpu.SemaphoreType.DMA,
            pltpu.SemaphoreType.DMA,
            pltpu.SemaphoreType.DMA,
        ],
    )
    def detile(tblT, o_hbm, tin_a, tin_b, tout_a, tout_b, tin64,
               is_a, is_b, os_a, os_b):
        wid = lax.axis_index("s") * NC + lax.axis_index("c")
        iota16 = lax.iota(jnp.int32, 16)

        def q_of(c):
            return c * NW + wid

        def fire(c, tin, sem):
            s = jnp.minimum(q_of(c) * QW, clamp)
            pltpu.async_copy(
                tblT.at[pl.ds(0, emb_dim), pl.ds(s, QW)], tin, sem)

        def drain_in(tin, sem):
            pltpu.make_async_copy(
                tblT.at[pl.ds(0, emb_dim), pl.ds(0, QW)], tin, sem).wait()

        def drain_out(tout, sem):
            pltpu.make_async_copy(
                tout, o_hbm.at[pl.ds(0, 128)], sem).wait()

        def transpose_quad(tin, tout):
            @plsc.parallel_loop(0, 32, 1, unroll=4)
            def step(h2):
                sl = h2 // 8
                rl0 = lax.rem(h2, 8)
                rlvec = rl0 * 16 + iota16
                rl_in = sl * 128 + rlvec
                rowvec = sl * 32 + lax.shift_right_logical(rlvec, 2)
                colbase = jnp.bitwise_and(rlvec, 3) * emb_dim
                for d0 in range(emb_dim):
                    dvec = jnp.bitwise_and(d0 + iota16, emb_dim - 1)
                    plsc.store_scatter(
                        tout,
                        [rowvec, colbase + dvec],
                        plsc.load_gather(tin, [dvec, rl_in]),
                    )

        def proc(c, tin, tout, osem):
            q = q_of(c)
            transpose_quad(tin, tout)

            @pl.when(q == tail_q)
            def _():
                pltpu.sync_copy(
                    tblT.at[pl.ds(0, emb_dim), pl.ds(n_full * 128, 64)],
                    tin64)

                @plsc.parallel_loop(0, 4, 1)
                def tstep(rl0):
                    rlvec = rl0 * 16 + iota16
                    rowvec = lax.shift_right_logical(rlvec, 2)
                    colbase = jnp.bitwise_and(rlvec, 3) * emb_dim
                    for d0 in range(emb_dim):
                        dvec = jnp.bitwise_and(d0 + iota16, emb_dim - 1)
                        plsc.store_scatter(
                            tout,
                            [rowvec, colbase + dvec],
                            plsc.load_gather(tin64, [dvec, rlvec]),
                        )

            pltpu.async_copy(tout, o_hbm.at[pl.ds(q * 128, 128)], osem)

        fire(0, tin_a, is_a)

        def pair(h, carry):
            c = 2 * h
            fire(c + 1, tin_b, is_b)
            drain_in(tin_a, is_a)

            @pl.when(h > 0)
            def _():
                drain_out(tout_a, os_a)

            proc(c, tin_a, tout_a, os_a)
            fire(c + 2, tin_a, is_a)
            drain_in(tin_b, is_b)

            @pl.when(h > 0)
            def _():
                drain_out(tout_b, os_b)

            proc(c + 1, tin_b, tout_b, os_b)
            return carry

        lax.fori_loop(0, (NQ_PW - 1) // 2, pair, 0)
        drain_in(tin_a, is_a)
        drain_out(tout_a, os_a)
        proc(NQ_PW - 1, tin_a, tout_a, os_a)
        drain_out(tout_a, os_a)
        drain_out(tout_b, os_b)

    return detile


@functools.lru_cache(maxsize=None)
def _make_emb(n_l: int, n_b: int, emb_dim: int):
    ntok = n_l * n_b
    assert emb_dim == 32 and n_b % 128 == 0 and ntok % (NW * SB) == 0
    pw = ntok // NW
    nsb = pw // SB
    assert nsb % 2 == 1 and nsb >= 3
    npairs = (nsb - 1) // 2
    sb_per_l = n_b // SB

    mesh = plsc.VectorSubcoreMesh(core_axis_name="c", subcore_axis_name="s")

    @functools.partial(
        pl.kernel,
        mesh=mesh,
        compiler_params=pltpu.CompilerParams(
            use_tc_tiling_on_sc=False, needs_layout_passes=False),
        out_type=jax.ShapeDtypeStruct(
            (n_l, emb_dim // 8, n_b // 128, 8, 128), jnp.float32),
        scratch_types=[
            pltpu.VMEM((pw,), jnp.int32),
            pltpu.VMEM((SB, emb_dim), jnp.float32),
            pltpu.VMEM((SB, emb_dim), jnp.float32),
            pltpu.VMEM((emb_dim // 8, SB // 128, 8, 128), jnp.float32),
            pltpu.SemaphoreType.DMA,
            pltpu.SemaphoreType.DMA,
            pltpu.SemaphoreType.DMA,
        ],
    )
    def emb(tbl, idx_hbm, p_hbm, idx_v, rows_a, rows_b, tr,
            gs_a, gs_b, osem):
        wid = lax.axis_index("s") * NC + lax.axis_index("c")
        base = wid * pw
        pltpu.sync_copy(idx_hbm.at[pl.ds(base, pw)], idx_v)
        sb0 = wid * nsb

        def fire(i, rows, sem):
            for k in range(SB // NIDX):
                pltpu.async_copy(
                    tbl.at[idx_v.at[pl.ds(i * SB + k * NIDX, NIDX)]],
                    rows.at[pl.ds(k * NIDX, NIDX)],
                    sem,
                )

        def drain(rows, sem):
            pltpu.make_async_copy(tbl.at[pl.ds(0, SB)], rows, sem).wait()

        def drain_out():
            for dg in range(emb_dim // 8):
                pltpu.make_async_copy(
                    tr.at[dg], p_hbm.at[0, dg, pl.ds(0, SB // 128)], osem,
                ).wait()

        def process(i, rows):
            @pl.when(i > 0)
            def _():
                drain_out()
            iota16 = lax.iota(jnp.int32, 16)

            @plsc.parallel_loop(0, SB // 16, 1, unroll=2)
            def tconv(j):
                ridx = j * 16 + iota16
                bgvec = jnp.full((16,), j // 8, jnp.int32)
                blvec = lax.rem(j, 8) * 16 + iota16
                for d0 in range(emb_dim):
                    dvec = jnp.bitwise_and(d0 + iota16, emb_dim - 1)
                    v = plsc.load_gather(rows, [ridx, dvec])
                    plsc.store_scatter(
                        tr,
                        [lax.shift_right_logical(dvec, 3), bgvec,
                         jnp.bitwise_and(dvec, 7), blvec],
                        v,
                    )
            sb = sb0 + i
            l = sb // sb_per_l
            bgbase = lax.rem(sb, sb_per_l) * (SB // 128)
            for dg in range(emb_dim // 8):
                pltpu.async_copy(
                    tr.at[dg],
                    p_hbm.at[l, dg, pl.ds(bgbase, SB // 128)],
                    osem,
                )

        fire(0, rows_a, gs_a)

        def pair(h, carry):
            i = 2 * h
            fire(i + 1, rows_b, gs_b)
            drain(rows_a, gs_a)
            process(i, rows_a)
            fire(i + 2, rows_a, gs_a)
            drain(rows_b, gs_b)
            process(i + 1, rows_b)
            return carry

        lax.fori_loop(0, npairs, pair, 0)
        drain(rows_a, gs_a)
        process(nsb - 1, rows_a)
        drain_out()

    return emb


def kernel(x, table):
    b, l = x.shape
    num_emb, emb_dim = table.shape
    idx = x.T.reshape(-1).astype(jnp.int32)
    o = _make_detile(num_emb, emb_dim)(table.T)
    tbl_lin = o.reshape(NROW_PAD, emb_dim)
    p = _make_emb(l, b, emb_dim)(tbl_lin, idx)
    return p.transpose(2, 4, 0, 1, 3).reshape(b, l, emb_dim)
